# baseline (speedup 1.0000x reference)
import functools

import jax
import jax.numpy as jnp
from jax import lax
from jax.experimental import pallas as pl
from jax.experimental.pallas import tpu as pltpu
from jax.experimental.pallas import tpu_sc as plsc

ENT_DIM = 128
REL_DIM = 64
MROW = ENT_DIM * REL_DIM
QROW = MROW // 4
GAMMA = 12.0
NC = 2
NS = 16
NW = NC * NS
LANES = 16


def _pack_body(x_ref, o_ref):
    lo = x_ref[:, :QROW].astype(jnp.bfloat16)
    hi = x_ref[:, QROW:].astype(jnp.bfloat16)
    lo32 = lax.bitcast_convert_type(lo, jnp.uint16).astype(jnp.uint32)
    hi32 = lax.bitcast_convert_type(hi, jnp.uint16).astype(jnp.uint32)
    o_ref[...] = lax.bitcast_convert_type(lo32 | (hi32 << 16), jnp.int32)


def _pack_mr_half(mr, half):
    rel_num = mr.shape[0]
    blk = 200
    return pl.pallas_call(
        _pack_body,
        grid=(rel_num // blk,),
        in_specs=[pl.BlockSpec((blk, MROW // 2), lambda i, h=half: (i, h))],
        out_specs=pl.BlockSpec((blk, QROW), lambda i: (i, 0)),
        out_shape=jax.ShapeDtypeStruct((rel_num, QROW), jnp.int32),
    )(mr)


@functools.lru_cache(maxsize=None)
def _make_sc_call(batch, phase):
    SPW = batch // NW
    CH = 16
    NCH = SPW // CH
    KC = ENT_DIM // LANES
    JC = REL_DIM // LANES

    mesh = plsc.VectorSubcoreMesh(core_axis_name="c", subcore_axis_name="s")

    if phase == 0:
        out_type = jax.ShapeDtypeStruct((batch,), jnp.float32)
    else:
        out_type = jax.ShapeDtypeStruct((batch,), jnp.float32)

    @functools.partial(
        pl.kernel,
        mesh=mesh,
        compiler_params=pltpu.CompilerParams(needs_layout_passes=False),
        out_type=out_type,
        scratch_types=[
            pltpu.VMEM((SPW,), jnp.int32),
            pltpu.VMEM((SPW,), jnp.int32),
            pltpu.VMEM((SPW,), jnp.int32),
            pltpu.VMEM((SPW, ENT_DIM), jnp.float32),
            pltpu.VMEM((SPW, ENT_DIM), jnp.float32),
            pltpu.VMEM((SPW, ENT_DIM), jnp.float32),
            pltpu.VMEM((CH, QROW), jnp.int32),
            pltpu.VMEM((CH, QROW), jnp.int32),
            pltpu.VMEM((SPW,), jnp.float32),
            pltpu.VMEM((SPW,), jnp.float32),
            pltpu.SemaphoreType.DMA,
            pltpu.SemaphoreType.DMA,
            pltpu.SemaphoreType.DMA,
            pltpu.SemaphoreType.DMA,
            pltpu.SemaphoreType.DMA,
        ],
    )
    def call(*refs):
        if phase == 0:
            (hidx_hbm, tidx_hbm, ridx_hbm, ent_hbm, rel_hbm, mr_hbm,
             out_hbm, hidx_v, tidx_v, ridx_v, d_rows, t_rows, rel_rows,
             mbuf0, mbuf1, nsq_v, aux_v, sem_h, sem_t, sem_r, sem_m0,
             sem_m1) = refs
            part_hbm = None
        else:
            (hidx_hbm, tidx_hbm, ridx_hbm, ent_hbm, rel_hbm, mr_hbm,
             part_hbm, out_hbm, hidx_v, tidx_v, ridx_v, d_rows, t_rows,
             rel_rows, mbuf0, mbuf1, nsq_v, aux_v, sem_h, sem_t, sem_r,
             sem_m0, sem_m1) = refs
        wid = lax.axis_index("s") * NC + lax.axis_index("c")
        base = wid * SPW

        pltpu.sync_copy(hidx_hbm.at[pl.ds(base, SPW)], hidx_v)
        pltpu.sync_copy(tidx_hbm.at[pl.ds(base, SPW)], tidx_v)
        pltpu.sync_copy(ridx_hbm.at[pl.ds(base, SPW)], ridx_v)

        def mcopy(c, buf, sem):
            return pltpu.make_async_copy(
                mr_hbm.at[ridx_v.at[pl.ds(c * CH, CH)]], buf, sem)

        mcopy(0, mbuf0, sem_m0).start()
        mcopy(1, mbuf1, sem_m1).start()

        cp_h = pltpu.async_copy(ent_hbm.at[hidx_v], d_rows, sem_h)
        cp_t = pltpu.async_copy(ent_hbm.at[tidx_v], t_rows, sem_t)
        cp_r = pltpu.async_copy(rel_hbm.at[ridx_v], rel_rows, sem_r)
        if phase == 1:
            pltpu.sync_copy(part_hbm.at[pl.ds(base, SPW)], aux_v)
        cp_h.wait()
        cp_t.wait()
        cp_r.wait()

        def dbody(s, carry):
            for c in range(KC):
                sl = pl.ds(c * LANES, LANES)
                d_rows[s, sl] = d_rows[s, sl] - t_rows[s, sl]
            return carry

        lax.fori_loop(0, SPW, dbody, 0)

        iota = lax.iota(jnp.int32, LANES)
        lane0 = iota == 0
        himask = jnp.full((LANES,), -65536, jnp.int32)
        jc0 = 0 if phase == 0 else 2

        def sample_body(sloc, c, buf):
            s = c * CH + sloc
            ddup = [plsc.pack(d_rows[s, pl.ds(kc * LANES, LANES)],
                              d_rows[s, pl.ds(kc * LANES, LANES)],
                              format=plsc.PackFormat.INTERLEAVED)
                    for kc in range(KC)]
            relch = [rel_rows[s, pl.ds((jc0 + jc) * LANES, LANES)]
                     for jc in range(2)]
            nsqs = [jnp.float32(0.0), jnp.float32(0.0)]
            for j in range(16):
                colb = j * KC * LANES
                p = [plsc.bitcast(buf[sloc, pl.ds(colb + kc * LANES, LANES)],
                                  jnp.bfloat16) * ddup[kc]
                     for kc in range(KC)]
                pt = (((p[0] + p[1]) + (p[2] + p[3]))
                      + ((p[4] + p[5]) + (p[6] + p[7])))
                mi = plsc.bitcast(pt, jnp.int32)
                ev = plsc.bitcast(lax.shift_left(mi, 16), jnp.float32)
                od = plsc.bitcast(lax.bitwise_and(mi, himask), jnp.float32)
                sj = jnp.sum(ev) + relch[0][j]
                sk = jnp.sum(od) + relch[1][j]
                nsqs[0] = nsqs[0] + sj * sj
                nsqs[1] = nsqs[1] + sk * sk
            plsc.store_scatter(nsq_v, [jnp.full((LANES,), s, jnp.int32)],
                               jnp.full((LANES,), nsqs[0] + nsqs[1],
                                        jnp.float32),
                               mask=lane0)

        def super_body(g, carry):
            for b, (buf, sem) in enumerate(((mbuf0, sem_m0), (mbuf1, sem_m1))):
                c = 2 * g + b
                mcopy(c, buf, sem).wait()

                def sb(sloc, carry2):
                    sample_body(sloc, c, buf)
                    return carry2

                lax.fori_loop(0, CH, sb, 0)

                @pl.when(c + 2 < NCH)
                def _():
                    mcopy(c + 2, buf, sem).start()
            return carry

        lax.fori_loop(0, NCH // 2, super_body, 0)

        if phase == 0:
            pltpu.sync_copy(nsq_v, out_hbm.at[pl.ds(base, SPW)])
        else:
            def sq_body(i, carry):
                sl = pl.ds(i * LANES, LANES)
                v = nsq_v[sl] + aux_v[sl]
                yi = lax.bitcast_convert_type(v, jnp.int32)
                yi = jnp.int32(0x5F3759DF) - lax.shift_right_logical(yi, 1)
                y = lax.bitcast_convert_type(yi, jnp.float32)
                for _ in range(3):
                    y = y * (1.5 - 0.5 * v * y * y)
                nsq_v[sl] = v * y - GAMMA
                return carry

            lax.fori_loop(0, SPW // LANES, sq_body, 0)
            pltpu.sync_copy(nsq_v, out_hbm.at[pl.ds(base, SPW)])

    return call


def kernel(pos_sample, ent_embd, rel_embd, mr):
    batch = pos_sample.shape[0]
    idx = pos_sample.astype(jnp.int32)
    h_idx = idx[:, 0]
    r_idx = idx[:, 1]
    t_idx = idx[:, 2]
    mr_a = _pack_mr_half(mr, 0)
    mr_b = _pack_mr_half(mr, 1)
    rel_pad = jnp.pad(rel_embd, ((0, 0), (0, ENT_DIM - REL_DIM)))
    part = _make_sc_call(batch, 0)(h_idx, t_idx, r_idx, ent_embd, rel_pad,
                                   mr_a)
    scores = _make_sc_call(batch, 1)(h_idx, t_idx, r_idx, ent_embd, rel_pad,
                                     mr_b, part)
    return scores.reshape(batch, 1)


# speedup vs baseline: 1.1665x; 1.1665x over previous
import functools

import jax
import jax.numpy as jnp
from jax import lax
from jax.experimental import pallas as pl
from jax.experimental.pallas import tpu as pltpu
from jax.experimental.pallas import tpu_sc as plsc

ENT_DIM = 128
REL_DIM = 64
MROW = ENT_DIM * REL_DIM
HROW = MROW // 2
GAMMA = 12.0
NC = 2
NS = 16
NW = NC * NS
LANES = 16


def _pack_body(x_ref, o_ref):
    lo = x_ref[:, :HROW].astype(jnp.bfloat16)
    hi = x_ref[:, HROW:].astype(jnp.bfloat16)
    lo32 = lax.bitcast_convert_type(lo, jnp.uint16).astype(jnp.uint32)
    hi32 = lax.bitcast_convert_type(hi, jnp.uint16).astype(jnp.uint32)
    o_ref[...] = lax.bitcast_convert_type(lo32 | (hi32 << 16), jnp.int32)


def _pack_mr(mr):
    rel_num = mr.shape[0]
    blk = 200
    return pl.pallas_call(
        _pack_body,
        grid=(rel_num // blk,),
        in_specs=[pl.BlockSpec((blk, MROW), lambda i: (i, 0))],
        out_specs=pl.BlockSpec((blk, HROW), lambda i: (i, 0)),
        out_shape=jax.ShapeDtypeStruct((rel_num, HROW), jnp.int32),
    )(mr)


@functools.lru_cache(maxsize=None)
def _make_sc_call(batch):
    SPW = batch // NW
    CH = 8
    NCH = SPW // CH
    KC = ENT_DIM // LANES
    JC = REL_DIM // LANES

    mesh = plsc.VectorSubcoreMesh(core_axis_name="c", subcore_axis_name="s")

    @functools.partial(
        pl.kernel,
        mesh=mesh,
        compiler_params=pltpu.CompilerParams(needs_layout_passes=False),
        out_type=jax.ShapeDtypeStruct((batch,), jnp.float32),
        scratch_types=[
            pltpu.VMEM((SPW,), jnp.int32),
            pltpu.VMEM((SPW,), jnp.int32),
            pltpu.VMEM((SPW,), jnp.int32),
            pltpu.VMEM((SPW, ENT_DIM), jnp.float32),
            pltpu.VMEM((SPW, ENT_DIM), jnp.float32),
            pltpu.VMEM((SPW, ENT_DIM), jnp.float32),
            pltpu.VMEM((CH, HROW), jnp.int32),
            pltpu.VMEM((CH, HROW), jnp.int32),
            pltpu.VMEM((SPW,), jnp.float32),
            pltpu.VMEM((SPW,), jnp.float32),
            pltpu.SemaphoreType.DMA,
            pltpu.SemaphoreType.DMA,
            pltpu.SemaphoreType.DMA,
            pltpu.SemaphoreType.DMA,
            pltpu.SemaphoreType.DMA,
        ],
    )
    def call(hidx_hbm, tidx_hbm, ridx_hbm, ent_hbm, rel_hbm, mr_hbm,
             out_hbm, hidx_v, tidx_v, ridx_v, d_rows, t_rows, rel_rows,
             mbuf0, mbuf1, nsq_v, scr_v, sem_h, sem_t, sem_r, sem_m0,
             sem_m1):
        wid = lax.axis_index("s") * NC + lax.axis_index("c")
        base = wid * SPW

        pltpu.sync_copy(hidx_hbm.at[pl.ds(base, SPW)], hidx_v)
        pltpu.sync_copy(tidx_hbm.at[pl.ds(base, SPW)], tidx_v)
        pltpu.sync_copy(ridx_hbm.at[pl.ds(base, SPW)], ridx_v)

        def mcopy(c, buf, sem):
            return pltpu.make_async_copy(
                mr_hbm.at[ridx_v.at[pl.ds(c * CH, CH)]], buf, sem)

        mcopy(0, mbuf0, sem_m0).start()
        mcopy(1, mbuf1, sem_m1).start()

        cp_h = pltpu.async_copy(ent_hbm.at[hidx_v], d_rows, sem_h)
        cp_t = pltpu.async_copy(ent_hbm.at[tidx_v], t_rows, sem_t)
        cp_r = pltpu.async_copy(rel_hbm.at[ridx_v], rel_rows, sem_r)
        cp_h.wait()
        cp_t.wait()
        cp_r.wait()

        def dbody(s, carry):
            for c in range(KC):
                sl = pl.ds(c * LANES, LANES)
                d_rows[s, sl] = d_rows[s, sl] - t_rows[s, sl]
            return carry

        lax.fori_loop(0, SPW, dbody, 0)

        iota = lax.iota(jnp.int32, LANES)
        lane0 = iota == 0
        himask = jnp.full((LANES,), -65536, jnp.int32)

        def sample_body(sloc, c, buf):
            s = c * CH + sloc
            ddup = [plsc.pack(d_rows[s, pl.ds(kc * LANES, LANES)],
                              d_rows[s, pl.ds(kc * LANES, LANES)],
                              format=plsc.PackFormat.INTERLEAVED)
                    for kc in range(KC)]
            relch = [rel_rows[s, pl.ds(jc * LANES, LANES)] for jc in range(JC)]
            nsqs = [jnp.float32(0.0), jnp.float32(0.0)]
            for j in range(32):
                colb = j * KC * LANES
                p = [plsc.bitcast(buf[sloc, pl.ds(colb + kc * LANES, LANES)],
                                  jnp.bfloat16) * ddup[kc]
                     for kc in range(KC)]
                pt = (((p[0] + p[1]) + (p[2] + p[3]))
                      + ((p[4] + p[5]) + (p[6] + p[7])))
                mi = plsc.bitcast(pt, jnp.int32)
                ev = plsc.bitcast(lax.shift_left(mi, 16), jnp.float32)
                od = plsc.bitcast(lax.bitwise_and(mi, himask), jnp.float32)
                sj = jnp.sum(ev) + relch[j // 16][j % 16]
                sk = jnp.sum(od) + relch[2 + j // 16][j % 16]
                nsqs[0] = nsqs[0] + sj * sj
                nsqs[1] = nsqs[1] + sk * sk
            plsc.store_scatter(nsq_v, [jnp.full((LANES,), s, jnp.int32)],
                               jnp.full((LANES,), nsqs[0] + nsqs[1],
                                        jnp.float32),
                               mask=lane0)

        def super_body(g, carry):
            for b, (buf, sem) in enumerate(((mbuf0, sem_m0), (mbuf1, sem_m1))):
                c = 2 * g + b
                mcopy(c, buf, sem).wait()

                def sb(sloc, carry2):
                    sample_body(sloc, c, buf)
                    return carry2

                lax.fori_loop(0, CH, sb, 0)

                @pl.when(c + 2 < NCH)
                def _():
                    mcopy(c + 2, buf, sem).start()
            return carry

        lax.fori_loop(0, NCH // 2, super_body, 0)

        def sq_body(i, carry):
            sl = pl.ds(i * LANES, LANES)
            v = nsq_v[sl]
            yi = lax.bitcast_convert_type(v, jnp.int32)
            yi = jnp.int32(0x5F3759DF) - lax.shift_right_logical(yi, 1)
            y = lax.bitcast_convert_type(yi, jnp.float32)
            for _ in range(3):
                y = y * (1.5 - 0.5 * v * y * y)
            scr_v[sl] = v * y - GAMMA
            return carry

        lax.fori_loop(0, SPW // LANES, sq_body, 0)

        pltpu.sync_copy(scr_v, out_hbm.at[pl.ds(base, SPW)])

    return call


def kernel(pos_sample, ent_embd, rel_embd, mr):
    batch = pos_sample.shape[0]
    idx = pos_sample.astype(jnp.int32)
    h_idx = idx[:, 0]
    r_idx = idx[:, 1]
    t_idx = idx[:, 2]
    mr_packed = _pack_mr(mr)
    rel_pad = jnp.pad(rel_embd, ((0, 0), (0, ENT_DIM - REL_DIM)))
    call = _make_sc_call(batch)
    scores = call(h_idx, t_idx, r_idx, ent_embd, rel_pad, mr_packed)
    return scores.reshape(batch, 1)


# speedup vs baseline: 1.1800x; 1.0116x over previous
import functools

import jax
import jax.numpy as jnp
from jax import lax
from jax.experimental import pallas as pl
from jax.experimental.pallas import tpu as pltpu
from jax.experimental.pallas import tpu_sc as plsc

ENT_DIM = 128
REL_DIM = 64
MROW = ENT_DIM * REL_DIM
HROW = MROW // 2
GAMMA = 12.0
NC = 2
NS = 16
NW = NC * NS
LANES = 16


def _pack_body(x_ref, o_ref):
    lo = lax.bitcast_convert_type(x_ref[:, :HROW], jnp.uint32)
    hi = lax.bitcast_convert_type(x_ref[:, HROW:], jnp.uint32)
    o_ref[...] = lax.bitcast_convert_type(
        (lo >> 16) | (hi & jnp.uint32(0xFFFF0000)), jnp.int32)


def _pack_mr(mr):
    rel_num = mr.shape[0]
    blk = 200
    return pl.pallas_call(
        _pack_body,
        grid=(rel_num // blk,),
        in_specs=[pl.BlockSpec((blk, MROW), lambda i: (i, 0))],
        out_specs=pl.BlockSpec((blk, HROW), lambda i: (i, 0)),
        out_shape=jax.ShapeDtypeStruct((rel_num, HROW), jnp.int32),
    )(mr)


@functools.lru_cache(maxsize=None)
def _make_sc_call(batch):
    SPW = batch // NW
    CH = 8
    NCH = SPW // CH
    KC = ENT_DIM // LANES
    JC = REL_DIM // LANES

    mesh = plsc.VectorSubcoreMesh(core_axis_name="c", subcore_axis_name="s")

    @functools.partial(
        pl.kernel,
        mesh=mesh,
        compiler_params=pltpu.CompilerParams(needs_layout_passes=False),
        out_type=jax.ShapeDtypeStruct((batch,), jnp.float32),
        scratch_types=[
            pltpu.VMEM((SPW,), jnp.int32),
            pltpu.VMEM((SPW,), jnp.int32),
            pltpu.VMEM((SPW,), jnp.int32),
            pltpu.VMEM((SPW, ENT_DIM), jnp.float32),
            pltpu.VMEM((SPW, ENT_DIM), jnp.float32),
            pltpu.VMEM((SPW, ENT_DIM), jnp.float32),
            pltpu.VMEM((CH, HROW), jnp.int32),
            pltpu.VMEM((CH, HROW), jnp.int32),
            pltpu.VMEM((SPW,), jnp.float32),
            pltpu.VMEM((SPW,), jnp.float32),
            pltpu.SemaphoreType.DMA,
            pltpu.SemaphoreType.DMA,
            pltpu.SemaphoreType.DMA,
            pltpu.SemaphoreType.DMA,
            pltpu.SemaphoreType.DMA,
        ],
    )
    def call(hidx_hbm, tidx_hbm, ridx_hbm, ent_hbm, rel_hbm, mr_hbm,
             out_hbm, hidx_v, tidx_v, ridx_v, d_rows, t_rows, rel_rows,
             mbuf0, mbuf1, nsq_v, scr_v, sem_h, sem_t, sem_r, sem_m0,
             sem_m1):
        wid = lax.axis_index("s") * NC + lax.axis_index("c")
        base = wid * SPW

        pltpu.sync_copy(hidx_hbm.at[pl.ds(base, SPW)], hidx_v)
        pltpu.sync_copy(tidx_hbm.at[pl.ds(base, SPW)], tidx_v)
        pltpu.sync_copy(ridx_hbm.at[pl.ds(base, SPW)], ridx_v)

        def mcopy(c, buf, sem):
            return pltpu.make_async_copy(
                mr_hbm.at[ridx_v.at[pl.ds(c * CH, CH)]], buf, sem)

        mcopy(0, mbuf0, sem_m0).start()
        mcopy(1, mbuf1, sem_m1).start()

        cp_h = pltpu.async_copy(ent_hbm.at[hidx_v], d_rows, sem_h)
        cp_t = pltpu.async_copy(ent_hbm.at[tidx_v], t_rows, sem_t)
        cp_r = pltpu.async_copy(rel_hbm.at[ridx_v], rel_rows, sem_r)
        cp_h.wait()
        cp_t.wait()
        cp_r.wait()

        def dbody(s, carry):
            for c in range(KC):
                sl = pl.ds(c * LANES, LANES)
                d_rows[s, sl] = d_rows[s, sl] - t_rows[s, sl]
            return carry

        lax.fori_loop(0, SPW, dbody, 0)

        iota = lax.iota(jnp.int32, LANES)
        lane0 = iota == 0
        himask = jnp.full((LANES,), -65536, jnp.int32)

        def sample_body(sloc, c, buf):
            s = c * CH + sloc
            ddup = [plsc.pack(d_rows[s, pl.ds(kc * LANES, LANES)],
                              d_rows[s, pl.ds(kc * LANES, LANES)],
                              format=plsc.PackFormat.INTERLEAVED)
                    for kc in range(KC)]
            relch = [rel_rows[s, pl.ds(jc * LANES, LANES)] for jc in range(JC)]
            nsqs = [jnp.float32(0.0), jnp.float32(0.0)]
            for j in range(32):
                colb = j * KC * LANES
                p = [plsc.bitcast(buf[sloc, pl.ds(colb + kc * LANES, LANES)],
                                  jnp.bfloat16) * ddup[kc]
                     for kc in range(KC)]
                pt = (((p[0] + p[1]) + (p[2] + p[3]))
                      + ((p[4] + p[5]) + (p[6] + p[7])))
                mi = plsc.bitcast(pt, jnp.int32)
                ev = plsc.bitcast(lax.shift_left(mi, 16), jnp.float32)
                od = plsc.bitcast(lax.bitwise_and(mi, himask), jnp.float32)
                sj = jnp.sum(ev) + relch[j // 16][j % 16]
                sk = jnp.sum(od) + relch[2 + j // 16][j % 16]
                nsqs[0] = nsqs[0] + sj * sj
                nsqs[1] = nsqs[1] + sk * sk
            plsc.store_scatter(nsq_v, [jnp.full((LANES,), s, jnp.int32)],
                               jnp.full((LANES,), nsqs[0] + nsqs[1],
                                        jnp.float32),
                               mask=lane0)

        def super_body(g, carry):
            for b, (buf, sem) in enumerate(((mbuf0, sem_m0), (mbuf1, sem_m1))):
                c = 2 * g + b
                mcopy(c, buf, sem).wait()

                def sb(sloc, carry2):
                    sample_body(sloc, c, buf)
                    return carry2

                lax.fori_loop(0, CH, sb, 0)

                @pl.when(c + 2 < NCH)
                def _():
                    mcopy(c + 2, buf, sem).start()
            return carry

        lax.fori_loop(0, NCH // 2, super_body, 0)

        def sq_body(i, carry):
            sl = pl.ds(i * LANES, LANES)
            v = nsq_v[sl]
            yi = lax.bitcast_convert_type(v, jnp.int32)
            yi = jnp.int32(0x5F3759DF) - lax.shift_right_logical(yi, 1)
            y = lax.bitcast_convert_type(yi, jnp.float32)
            for _ in range(3):
                y = y * (1.5 - 0.5 * v * y * y)
            scr_v[sl] = v * y - GAMMA
            return carry

        lax.fori_loop(0, SPW // LANES, sq_body, 0)

        pltpu.sync_copy(scr_v, out_hbm.at[pl.ds(base, SPW)])

    return call


def kernel(pos_sample, ent_embd, rel_embd, mr):
    batch = pos_sample.shape[0]
    idx = pos_sample.astype(jnp.int32)
    h_idx = idx[:, 0]
    r_idx = idx[:, 1]
    t_idx = idx[:, 2]
    mr_packed = _pack_mr(mr)
    rel_pad = jnp.pad(rel_embd, ((0, 0), (0, ENT_DIM - REL_DIM)))
    call = _make_sc_call(batch)
    scores = call(h_idx, t_idx, r_idx, ent_embd, rel_pad, mr_packed)
    return scores.reshape(batch, 1)
